# baseline (speedup 1.0000x reference)
import jax
import jax.numpy as jnp
from jax.experimental import pallas as pl
from jax.experimental.pallas import tpu as pltpu

INTERPRET = False

_TB = 512
_SUB = 64
_LANES = 128


def _dist_argmin_kernel(z_ref, cb_ref, z2_ref, c2_ref, idx_ref, loss_ref,
                        mm_a, mm_b):
    pid = pl.program_id(0)
    tb, d = z_ref.shape
    kc = cb_ref.shape[0]
    nstrip = kc // _LANES
    zb2 = z_ref[...] * 2.0
    cb = cb_ref[...]
    lane_f = jax.lax.broadcasted_iota(
        jnp.int32, (_SUB, _LANES), 1).astype(jnp.float32)

    total = jnp.zeros((1, 1), jnp.float32)
    for sub in range(tb // _SUB):
        zs = zb2[sub * _SUB:(sub + 1) * _SUB, :]
        z2s = z2_ref[pl.ds(sub * _SUB, _SUB), :]
        mm_ref = mm_a if sub % 2 == 0 else mm_b
        mm_ref[...] = jax.lax.dot_general(
            zs, cb, (((1,), (1,)), ((), ())),
            preferred_element_type=jnp.float32)

        def strip_body(s, carry):
            rv, ri = carry
            c2s = c2_ref[:, pl.ds(s * _LANES, _LANES)]
            ms = mm_ref[:, pl.ds(s * _LANES, _LANES)]
            dist = (z2s + c2s) - ms
            cmp = dist < rv
            rv = jnp.minimum(dist, rv)
            sf = jax.lax.convert_element_type(s, jnp.float32)
            ri = jnp.where(cmp, sf, ri)
            return rv, ri

        init = (jnp.full((_SUB, _LANES), jnp.inf, jnp.float32),
                jnp.zeros((_SUB, _LANES), jnp.float32))
        rv, ri = jax.lax.fori_loop(0, nstrip, strip_body, init, unroll=4)

        val = jnp.min(rv, axis=1, keepdims=True)
        idxf = ri * jnp.float32(_LANES) + lane_f
        cand = jnp.where(rv == val, idxf, jnp.float32(2 ** 24))
        idx = jnp.min(cand, axis=1, keepdims=True)
        idx_ref[pl.ds(sub * _SUB, _SUB), :] = idx.astype(jnp.int32)
        total += jnp.sum(val).reshape(1, 1)

    @pl.when(pid == 0)
    def _():
        loss_ref[...] = jnp.zeros((1, 1), jnp.float32)

    loss_ref[...] += total

    @pl.when(pid == pl.num_programs(0) - 1)
    def _():
        ntok_total = pl.num_programs(0) * tb
        m = loss_ref[...] / jnp.float32(ntok_total * d)
        loss_ref[...] = m + 0.25 * m


def kernel(z, codebook):
    b, l, d = z.shape
    kc = codebook.shape[0]
    ntok = b * l
    flat_z = z.reshape(-1, d)
    z2 = jnp.sum(flat_z ** 2, axis=-1, keepdims=True)
    c2 = jnp.sum(codebook ** 2, axis=-1, keepdims=True).T
    idx_flat, loss = pl.pallas_call(
        _dist_argmin_kernel,
        grid=(ntok // _TB,),
        in_specs=[
            pl.BlockSpec((_TB, d), lambda i: (i, 0)),
            pl.BlockSpec((kc, d), lambda i: (0, 0)),
            pl.BlockSpec((_TB, 1), lambda i: (i, 0)),
            pl.BlockSpec((1, kc), lambda i: (0, 0)),
        ],
        out_specs=[
            pl.BlockSpec((_TB, 1), lambda i: (i, 0)),
            pl.BlockSpec((1, 1), lambda i: (0, 0)),
        ],
        out_shape=[
            jax.ShapeDtypeStruct((ntok, 1), jnp.int32),
            jax.ShapeDtypeStruct((1, 1), jnp.float32),
        ],
        scratch_shapes=[
            pltpu.VMEM((_SUB, kc), jnp.float32),
            pltpu.VMEM((_SUB, kc), jnp.float32),
        ],
        interpret=INTERPRET,
    )(flat_z, codebook, z2, c2)
    idx_flat = idx_flat.reshape(-1)
    quantized = jnp.take(codebook, idx_flat, axis=0)
    qst = flat_z + (quantized - flat_z)
    return qst.reshape(b, l, d), idx_flat.reshape(b, l), loss[0, 0]


# speedup vs baseline: 2.6119x; 2.6119x over previous
import jax
import jax.numpy as jnp
from jax.experimental import pallas as pl
from jax.experimental.pallas import tpu as pltpu

INTERPRET = False

_TB = 512
_CHUNK = 4096


def _dist_argmin_kernel(z_ref, cb_ref, z2_ref, c2_ref, idx_ref, loss_ref,
                        mm_a, mm_b, iota_ref):
    pid = pl.program_id(0)
    tb, d = z_ref.shape
    kc = cb_ref.shape[0]
    zb2 = z_ref[...] * 2.0
    z2 = z2_ref[...]

    @pl.when(pid == 0)
    def _():
        iota_ref[...] = jax.lax.broadcasted_iota(
            jnp.int32, (tb, _CHUNK), 1).astype(jnp.float32)
        loss_ref[...] = jnp.zeros((1, 1), jnp.float32)

    mm_refs = (mm_a, mm_b)
    for c in range(kc // _CHUNK):
        cb_chunk = cb_ref[c * _CHUNK:(c + 1) * _CHUNK, :]
        mm_refs[c][...] = jax.lax.dot_general(
            zb2, cb_chunk, (((1,), (1,)), ((), ())),
            preferred_element_type=jnp.float32)

    best_val = None
    for c in range(kc // _CHUNK):
        mm_ref = mm_refs[c]
        c2_chunk = c2_ref[:, c * _CHUNK:(c + 1) * _CHUNK]
        mm_ref[...] = (z2 + c2_chunk) - mm_ref[...]
        dist = mm_ref[...]
        cval = jnp.min(dist, axis=1, keepdims=True)
        cand = jnp.where(dist == cval, iota_ref[...], jnp.float32(2 ** 24))
        cidx = jnp.min(cand, axis=1, keepdims=True) + jnp.float32(c * _CHUNK)
        if best_val is None:
            best_val, best_idx = cval, cidx
        else:
            upd = cval < best_val
            best_val = jnp.where(upd, cval, best_val)
            best_idx = jnp.where(upd, cidx, best_idx)

    idx_ref[...] = best_idx.astype(jnp.int32)
    loss_ref[...] += jnp.sum(best_val).reshape(1, 1)

    @pl.when(pid == pl.num_programs(0) - 1)
    def _():
        ntok_total = pl.num_programs(0) * tb
        m = loss_ref[...] / jnp.float32(ntok_total * d)
        loss_ref[...] = m + 0.25 * m


def kernel(z, codebook):
    b, l, d = z.shape
    kc = codebook.shape[0]
    ntok = b * l
    flat_z = z.reshape(-1, d)
    z2 = jnp.sum(flat_z ** 2, axis=-1, keepdims=True)
    c2 = jnp.sum(codebook ** 2, axis=-1, keepdims=True).T
    idx_flat, loss = pl.pallas_call(
        _dist_argmin_kernel,
        grid=(ntok // _TB,),
        in_specs=[
            pl.BlockSpec((_TB, d), lambda i: (i, 0)),
            pl.BlockSpec((kc, d), lambda i: (0, 0)),
            pl.BlockSpec((_TB, 1), lambda i: (i, 0)),
            pl.BlockSpec((1, kc), lambda i: (0, 0)),
        ],
        out_specs=[
            pl.BlockSpec((_TB, 1), lambda i: (i, 0)),
            pl.BlockSpec((1, 1), lambda i: (0, 0)),
        ],
        out_shape=[
            jax.ShapeDtypeStruct((ntok, 1), jnp.int32),
            jax.ShapeDtypeStruct((1, 1), jnp.float32),
        ],
        scratch_shapes=[
            pltpu.VMEM((_TB, _CHUNK), jnp.float32),
            pltpu.VMEM((_TB, _CHUNK), jnp.float32),
            pltpu.VMEM((_TB, _CHUNK), jnp.float32),
        ],
        interpret=INTERPRET,
    )(flat_z, codebook, z2, c2)
    idx_flat = idx_flat.reshape(-1)
    quantized = jnp.take(codebook, idx_flat, axis=0)
    qst = flat_z + (quantized - flat_z)
    return qst.reshape(b, l, d), idx_flat.reshape(b, l), loss[0, 0]
